# baseline (speedup 1.0000x reference)
import functools

import jax
import jax.numpy as jnp
from jax import lax
from jax.experimental import pallas as pl
from jax.experimental.pallas import tpu as pltpu
from jax.experimental.pallas import tpu_sc as plsc

_V = 1000000
_B = 16384
_D = 16
_NC = 2
_NS = 16
_NW = _NC * _NS
_BPW = _B // _NW
_CHUNK = 128
_NCH = _BPW // _CHUNK
_VC = 65536
_REG = 0.0001

_LO = 524288
_HI = 983040
_S = _HI - _LO
_SPW = _S // _NW
_TPC = 8
_LPC = _TPC * 128
_NCHK = _SPW // _LPC


def _tc_body(pt, qt, rt, w, spq, sr, np_, nq_, nr):
    p = pt[...]
    q = qt[...]
    r = rt[...]
    ww = w[...]
    dn = (((1,), (0,)), ((), ()))
    f32 = jnp.float32
    mp = lax.dot_general(ww, p, dn, preferred_element_type=f32)
    mq = lax.dot_general(ww, q, dn, preferred_element_type=f32)
    mr = lax.dot_general(ww, r, dn, preferred_element_type=f32)
    m2p = lax.dot_general(ww, p * p, dn, preferred_element_type=f32)
    m2q = lax.dot_general(ww, q * q, dn, preferred_element_type=f32)
    m2r = lax.dot_general(ww, r * r, dn, preferred_element_type=f32)
    spq[...] = mp[0] + mq[1]
    sr[...] = mr[2]
    np_[...] = m2p[3]
    nq_[...] = m2q[3]
    nr[...] = m2r[3]


def _tc_block2(c):
    return (0, jnp.where(c < 8, c, 15))


def _tc_block1(c):
    return (jnp.where(c < 8, c, 15),)


@functools.partial(
    pl.kernel,
    out_type=[jax.ShapeDtypeStruct((_S,), jnp.float32)] * 5,
    mesh=plsc.VectorSubcoreMesh(core_axis_name="c", subcore_axis_name="s"),
    scratch_types=[
        pltpu.VMEM((2 * 2 * _TPC * 8, 128), jnp.float32),
        pltpu.VMEM((2 * 2 * _TPC * 8, 128), jnp.float32),
        pltpu.VMEM((2 * 2 * _TPC * 8, 128), jnp.float32),
        pltpu.VMEM((8, _D), jnp.float32),
        pltpu.VMEM((_LPC,), jnp.float32),
        pltpu.VMEM((_LPC,), jnp.float32),
        pltpu.VMEM((_LPC,), jnp.float32),
        pltpu.VMEM((_LPC,), jnp.float32),
        pltpu.VMEM((_LPC,), jnp.float32),
        pltpu.SemaphoreType.DMA,
        pltpu.SemaphoreType.DMA,
    ],
)
def _sc_stream(pt_hbm, qt_hbm, rt_hbm, fcs_hbm,
               spq_hbm, sr_hbm, np_hbm, nq_hbm, nr_hbm,
               pv, qv, rv, fcv, st0, st1, st2, st3, st4, semA, semB):
    wid = lax.axis_index("s") * _NC + lax.axis_index("c")
    wbase = _LO + wid * _SPW

    pltpu.sync_copy(fcs_hbm, fcv)
    fps = [fcv[0][d] for d in range(_D)]
    fqs = [fcv[1][d] for d in range(_D)]
    frs = [fcv[2][d] for d in range(_D)]

    sems = (semA, semB)

    def fire(chunk, buf):
        ch = jnp.minimum(chunk, _NCHK - 1)
        base = wbase + ch * _LPC
        for tr in range(2):
            rows = pl.ds(tr * 8, 8)
            for t in range(_TPC):
                lanes = pl.ds(pl.multiple_of(base + t * 128, 128), 128)
                dst = pl.ds(((buf * 2 + tr) * _TPC + t) * 8, 8)
                pltpu.async_copy(pt_hbm.at[rows, lanes], pv.at[dst], sems[buf])
                pltpu.async_copy(qt_hbm.at[rows, lanes], qv.at[dst], sems[buf])
                pltpu.async_copy(rt_hbm.at[rows, lanes], rv.at[dst], sems[buf])

    def drain(buf):
        rows = pl.ds(0, 8)
        lanes = pl.ds(0, 128)
        dst = pl.ds(0, 8)
        for _ in range(2 * _TPC):
            pltpu.make_async_copy(
                pt_hbm.at[rows, lanes], pv.at[dst], sems[buf]).wait()
            pltpu.make_async_copy(
                qt_hbm.at[rows, lanes], qv.at[dst], sems[buf]).wait()
            pltpu.make_async_copy(
                rt_hbm.at[rows, lanes], rv.at[dst], sems[buf]).wait()

    def compute(chunk, buf):
        def g_body(g, _):
            ds16 = pl.ds(g * _D, _D)
            zero = jnp.zeros((_D,), jnp.float32)
            for t in range(_TPC):
                aspq = zero
                asr = zero
                anp = zero
                anq = zero
                anr = zero
                for tr in range(2):
                    base_row = ((buf * 2 + tr) * _TPC + t) * 8
                    for s in range(8):
                        d = tr * 8 + s
                        p = pv[base_row + s, ds16]
                        q = qv[base_row + s, ds16]
                        r = rv[base_row + s, ds16]
                        aspq = aspq + p * fps[d] + q * fqs[d]
                        asr = asr + r * frs[d]
                        anp = anp + p * p
                        anq = anq + q * q
                        anr = anr + r * r
                o = pl.ds(t * 128 + g * _D, _D)
                st0[o] = aspq
                st1[o] = asr
                st2[o] = anp
                st3[o] = anq
                st4[o] = anr
            return 0

        lax.fori_loop(0, 128 // _D, g_body, 0)

        obase = pl.multiple_of(wid * _SPW + chunk * _LPC, 8)
        dst = pl.ds(obase, _LPC)
        pltpu.sync_copy(st0, spq_hbm.at[dst])
        pltpu.sync_copy(st1, sr_hbm.at[dst])
        pltpu.sync_copy(st2, np_hbm.at[dst])
        pltpu.sync_copy(st3, nq_hbm.at[dst])
        pltpu.sync_copy(st4, nr_hbm.at[dst])

    fire(0, 0)

    def pair(i, _):
        c0 = i * 2
        fire(c0 + 1, 1)
        drain(0)
        compute(c0, 0)
        fire(c0 + 2, 0)
        drain(1)
        compute(c0 + 1, 1)
        return 0

    lax.fori_loop(0, _NCHK // 2, pair, 0)
    drain(0)


@functools.partial(
    pl.kernel,
    out_type=[
        jax.ShapeDtypeStruct((_B,), jnp.float32),
        jax.ShapeDtypeStruct((_NW, 3, _D), jnp.float32),
    ],
    mesh=plsc.VectorSubcoreMesh(core_axis_name="c", subcore_axis_name="s"),
    scratch_types=[
        pltpu.VMEM((_NCH, _CHUNK), jnp.int32),
        pltpu.VMEM((_NCH, _CHUNK), jnp.int32),
        pltpu.VMEM((_NCH, _CHUNK), jnp.int32),
        pltpu.VMEM((_NCH, _CHUNK), jnp.int32),
        pltpu.VMEM((10 * _BPW,), jnp.float32),
        pltpu.VMEM((_BPW,), jnp.float32),
        pltpu.VMEM((3, _D), jnp.float32),
        pltpu.SemaphoreType.DMA,
    ],
)
def _sc_gather(ps_hbm, rs_hbm,
               spqA, srA, npA, nqA, nrA,
               spqB, srB, npB, nqB, nrB,
               out_hbm, part_hbm,
               idxp, idxr, idxpB, idxrB, gv, outv, partv, sem):
    wid = lax.axis_index("s") * _NC + lax.axis_index("c")
    base = wid * _BPW

    pltpu.sync_copy(ps_hbm.at[pl.ds(wid * _NCH, _NCH)], idxp)
    pltpu.sync_copy(rs_hbm.at[pl.ds(wid * _NCH, _NCH)], idxr)

    def reb(j, _):
        for g in range(_CHUNK // _D):
            ds16 = pl.ds(g * _D, _D)
            vp = idxp[j, ds16]
            vr = idxr[j, ds16]
            mp = (vp >= _LO) & (vp < _HI)
            mr = (vr >= _LO) & (vr < _HI)
            idxpB[j, ds16] = jnp.where(mp, vp - _LO, vp >> 2)
            idxrB[j, ds16] = jnp.where(mr, vr - _LO, vr >> 2)
        return 0

    lax.fori_loop(0, _NCH, reb, 0)

    copies = []
    for j in range(_NCH):
        ip, ir = idxp.at[j], idxr.at[j]
        ipB, irB = idxpB.at[j], idxrB.at[j]
        for k, (arr, idx) in enumerate((
                (spqA, ip), (npA, ip), (nqA, ip), (srA, ir), (nrA, ir),
                (spqB, ipB), (npB, ipB), (nqB, ipB), (srB, irB), (nrB, irB))):
            dst = pl.ds(k * _BPW + j * _CHUNK, _CHUNK)
            copies.append(pltpu.async_copy(arr.at[idx], gv.at[dst], sem))
    for c in copies:
        c.wait()

    zero = jnp.zeros((_D,), jnp.float32)

    def blk(i, carry):
        ap, aq, ar = carry
        ds16 = pl.ds(i * _D, _D)
        j = i // (_CHUNK // _D)
        g = pl.ds((i % (_CHUNK // _D)) * _D, _D)
        vp = idxp[j, g]
        vr = idxr[j, g]
        mp = (vp >= _LO) & (vp < _HI)
        mr = (vr >= _LO) & (vr < _HI)

        def gval(k):
            return gv[pl.ds(k * _BPW + i * _D, _D)]

        spq = jnp.where(mp, gval(5), gval(0))
        sr = jnp.where(mr, gval(8), gval(3))
        npv = jnp.where(mp, gval(6), gval(1))
        nqv = jnp.where(mp, gval(7), gval(2))
        nrv = jnp.where(mr, gval(9), gval(4))
        outv[ds16] = spq + sr
        return (ap + npv, aq + nqv, ar + nrv)

    ap, aq, ar = lax.fori_loop(0, _BPW // _D, blk, (zero, zero, zero))
    partv[0] = ap
    partv[1] = aq
    partv[2] = ar

    pltpu.sync_copy(outv, out_hbm.at[pl.ds(base, _BPW)])
    pltpu.sync_copy(partv, part_hbm.at[wid])


def kernel(ps, qs, rs, P_table, Q_table, R_table, wp, wq, wr, fc_w):
    del qs
    fc = fc_w[0].astype(jnp.float32)
    fc_c = fc / jnp.maximum(jnp.sqrt(jnp.sum(fc * fc)), 1.0)

    def _cw(w):
        s = w[0, 0]
        return s / jnp.maximum(jnp.abs(s), 1.0)

    fcs = jnp.concatenate([
        jnp.stack([fc_c * _cw(wp), fc_c * _cw(wq), fc_c * _cw(wr),
                   jnp.ones((_D,), jnp.float32)]),
        jnp.zeros((4, _D), jnp.float32),
    ])

    Pt, Qt, Rt = P_table.T, Q_table.T, R_table.T

    scB = _sc_stream(Pt, Qt, Rt, fcs)

    tcA = pl.pallas_call(
        _tc_body,
        grid=(9,),
        in_specs=[
            pl.BlockSpec((_D, _VC), _tc_block2),
            pl.BlockSpec((_D, _VC), _tc_block2),
            pl.BlockSpec((_D, _VC), _tc_block2),
            pl.BlockSpec((8, _D), lambda c: (0, 0)),
        ],
        out_specs=[pl.BlockSpec((_VC,), _tc_block1)] * 5,
        out_shape=[jax.ShapeDtypeStruct((_V,), jnp.float32)] * 5,
    )(Pt, Qt, Rt, fcs)

    ps2 = ps.astype(jnp.int32).reshape(_NW * _NCH, _CHUNK)
    rs2 = rs.astype(jnp.int32).reshape(_NW * _NCH, _CHUNK)

    out, parts = _sc_gather(ps2, rs2, *tcA, *scB)

    inferences = out.reshape(_B, 1)
    sums = parts.sum(axis=(0, 2))
    regs = _REG * (jnp.sqrt(sums[0]) + jnp.sqrt(sums[1]) + jnp.sqrt(sums[2]))
    return (inferences, regs)


# speedup vs baseline: 1.1465x; 1.1465x over previous
import functools

import jax
import jax.numpy as jnp
from jax import lax
from jax.experimental import pallas as pl
from jax.experimental.pallas import tpu as pltpu
from jax.experimental.pallas import tpu_sc as plsc

_V = 1000000
_B = 16384
_D = 16
_NC = 2
_NS = 16
_NW = _NC * _NS
_BPW = _B // _NW
_CHUNK = 128
_NCH = _BPW // _CHUNK
_VC = 65536
_GA = -(-_V // _VC)
_REG = 0.0001


def _tc_body(pt, qt, rt, w, spq, sr, np_, nq_, nr):
    p = pt[...]
    q = qt[...]
    r = rt[...]
    ww = w[...]
    dn = (((1,), (0,)), ((), ()))
    f32 = jnp.float32
    mp = lax.dot_general(ww, p, dn, preferred_element_type=f32)
    mq = lax.dot_general(ww, q, dn, preferred_element_type=f32)
    mr = lax.dot_general(ww, r, dn, preferred_element_type=f32)
    m2p = lax.dot_general(ww, p * p, dn, preferred_element_type=f32)
    m2q = lax.dot_general(ww, q * q, dn, preferred_element_type=f32)
    m2r = lax.dot_general(ww, r * r, dn, preferred_element_type=f32)
    spq[...] = mp[0] + mq[1]
    sr[...] = mr[2]
    np_[...] = m2p[3]
    nq_[...] = m2q[3]
    nr[...] = m2r[3]


@functools.partial(
    pl.kernel,
    out_type=[
        jax.ShapeDtypeStruct((_B,), jnp.float32),
        jax.ShapeDtypeStruct((_NW, 3, _D), jnp.float32),
    ],
    mesh=plsc.VectorSubcoreMesh(core_axis_name="c", subcore_axis_name="s"),
    scratch_types=[
        pltpu.VMEM((_NCH, _CHUNK), jnp.int32),
        pltpu.VMEM((_NCH, _CHUNK), jnp.int32),
        pltpu.VMEM((_BPW,), jnp.float32),
        pltpu.VMEM((_BPW,), jnp.float32),
        pltpu.VMEM((_BPW,), jnp.float32),
        pltpu.VMEM((_BPW,), jnp.float32),
        pltpu.VMEM((_BPW,), jnp.float32),
        pltpu.VMEM((_BPW,), jnp.float32),
        pltpu.VMEM((3, _D), jnp.float32),
        pltpu.SemaphoreType.DMA,
    ],
)
def _sc_body(ps_hbm, rs_hbm, spq_hbm, sr_hbm, np_hbm, nq_hbm, nr_hbm,
             out_hbm, part_hbm,
             idxp_v, idxr_v, spq_v, sr_v, np_v, nq_v, nr_v, outv, partv, sem):
    wid = lax.axis_index("s") * _NC + lax.axis_index("c")
    base = wid * _BPW

    pltpu.sync_copy(ps_hbm.at[pl.ds(wid * _NCH, _NCH)], idxp_v)
    pltpu.sync_copy(rs_hbm.at[pl.ds(wid * _NCH, _NCH)], idxr_v)

    copies = []
    for j in range(_NCH):
        dst = pl.ds(j * _CHUNK, _CHUNK)
        ip = idxp_v.at[j]
        ir = idxr_v.at[j]
        copies.append(pltpu.async_copy(spq_hbm.at[ip], spq_v.at[dst], sem))
        copies.append(pltpu.async_copy(np_hbm.at[ip], np_v.at[dst], sem))
        copies.append(pltpu.async_copy(nq_hbm.at[ip], nq_v.at[dst], sem))
        copies.append(pltpu.async_copy(sr_hbm.at[ir], sr_v.at[dst], sem))
        copies.append(pltpu.async_copy(nr_hbm.at[ir], nr_v.at[dst], sem))
    for c in copies:
        c.wait()

    zero = jnp.zeros((_D,), jnp.float32)

    def blk(i, carry):
        ap, aq, ar = carry
        ds = pl.ds(i * _D, _D)
        outv[ds] = spq_v[ds] + sr_v[ds]
        return (ap + np_v[ds], aq + nq_v[ds], ar + nr_v[ds])

    ap, aq, ar = lax.fori_loop(0, _BPW // _D, blk, (zero, zero, zero))
    partv[0] = ap
    partv[1] = aq
    partv[2] = ar

    pltpu.sync_copy(outv, out_hbm.at[pl.ds(base, _BPW)])
    pltpu.sync_copy(partv, part_hbm.at[wid])


def kernel(ps, qs, rs, P_table, Q_table, R_table, wp, wq, wr, fc_w):
    del qs
    fc = fc_w[0].astype(jnp.float32)
    fc_c = fc / jnp.maximum(jnp.sqrt(jnp.sum(fc * fc)), 1.0)

    def _cw(w):
        s = w[0, 0]
        return s / jnp.maximum(jnp.abs(s), 1.0)

    fcs = jnp.concatenate([
        jnp.stack([fc_c * _cw(wp), fc_c * _cw(wq), fc_c * _cw(wr),
                   jnp.ones((_D,), jnp.float32)]),
        jnp.zeros((4, _D), jnp.float32),
    ])

    spq, sr, np_, nq_, nr = pl.pallas_call(
        _tc_body,
        grid=(_GA,),
        in_specs=[
            pl.BlockSpec((_D, _VC), lambda c: (0, c)),
            pl.BlockSpec((_D, _VC), lambda c: (0, c)),
            pl.BlockSpec((_D, _VC), lambda c: (0, c)),
            pl.BlockSpec((8, _D), lambda c: (0, 0)),
        ],
        out_specs=[pl.BlockSpec((_VC,), lambda c: (c,))] * 5,
        out_shape=[jax.ShapeDtypeStruct((_V,), jnp.float32)] * 5,
    )(P_table.T, Q_table.T, R_table.T, fcs)

    ps2 = ps.astype(jnp.int32).reshape(_NW * _NCH, _CHUNK)
    rs2 = rs.astype(jnp.int32).reshape(_NW * _NCH, _CHUNK)

    out, parts = _sc_body(ps2, rs2, spq, sr, np_, nq_, nr)

    inferences = out.reshape(_B, 1)
    sums = parts.sum(axis=(0, 2))
    regs = _REG * (jnp.sqrt(sums[0]) + jnp.sqrt(sums[1]) + jnp.sqrt(sums[2]))
    return (inferences, regs)
